# baseline (speedup 1.0000x reference)
import functools

import jax
import jax.numpy as jnp
from jax import lax
from jax.experimental import pallas as pl
from jax.experimental.pallas import tpu as pltpu
from jax.experimental.pallas import tpu_sc as plsc

EMBED_DIM = 64
NUM_CLS = 16
SEQ = 200
CW = 128
TAIL = SEQ - CW
LANES = 16
NQ = EMBED_DIM // LANES


@functools.cache
def _sc_pool(batch):
  info = plsc.get_sparse_core_info()
  num_workers = info.num_cores * info.num_subcores
  bpw = batch // num_workers
  nbuf = 4
  mesh = plsc.VectorSubcoreMesh(core_axis_name="c", subcore_axis_name="s")

  @functools.partial(
      pl.kernel,
      out_type=jax.ShapeDtypeStruct((batch, EMBED_DIM), jnp.float32),
      mesh=mesh,
      scratch_types=[
          pltpu.VMEM((bpw, CW), jnp.int32),
          pltpu.VMEM((bpw, CW), jnp.int32),
          pltpu.VMEM((nbuf, CW, EMBED_DIM), jnp.float32),
          pltpu.VMEM((bpw, EMBED_DIM), jnp.float32),
          pltpu.SemaphoreType.DMA,
      ],
      compiler_params=pltpu.CompilerParams(use_tc_tiling_on_sc=False),
  )
  def sc_pool(
      ids_a_hbm, ids_b_hbm, emb_hbm, out_hbm, idx_a, idx_b, rows_v, pooled_v,
      sem,
  ):
    wid = lax.axis_index("s") * info.num_cores + lax.axis_index("c")
    pltpu.sync_copy(ids_a_hbm.at[pl.ds(wid * bpw, bpw)], idx_a)
    pltpu.sync_copy(ids_b_hbm.at[pl.ds(wid * bpw, bpw)], idx_b)

    def dma(row, parity, buf):
      if parity == 0:
        return pltpu.make_async_copy(
            emb_hbm.at[idx_a.at[row]], rows_v.at[buf], sem
        )
      return pltpu.make_async_copy(
          emb_hbm.at[idx_b.at[row, pl.ds(0, TAIL)]],
          rows_v.at[buf, pl.ds(0, TAIL)],
          sem,
      )

    for c in range(nbuf - 1):
      dma(c // 2, c % 2, c).start()

    zeros = (jnp.zeros((LANES,), jnp.float32),) * NQ

    def reduce_span(buf, hi, acc):
      def body(r, a):
        return tuple(
            a[q] + rows_v[buf, r, pl.ds(q * LANES, LANES)] for q in range(NQ)
        )

      return lax.fori_loop(0, hi, body, acc, unroll=4)

    def outer(g, carry):
      row0 = g * (nbuf // 2)
      for c in range(nbuf):
        parity = c % 2
        row = row0 + c // 2
        nxt_row = row0 + (c + nbuf - 1) // 2

        @pl.when(nxt_row < bpw)
        def _():
          dma(nxt_row, (c + nbuf - 1) % 2, (c + nbuf - 1) % nbuf).start()

        dma(row, parity, c).wait()
        if parity == 0:
          acc = reduce_span(c, CW, zeros)
        else:
          acc = reduce_span(c, TAIL, acc)
          for q in range(NQ):
            pooled_v[row, pl.ds(q * LANES, LANES)] = acc[q] * (1.0 / SEQ)
      return carry

    lax.fori_loop(0, 2 * bpw // nbuf, outer, 0)
    pltpu.sync_copy(pooled_v, out_hbm.at[pl.ds(wid * bpw, bpw)])

  return sc_pool


def _tc_head(pooled, w_t, bias):
  def body(p_ref, w_ref, b_ref, o_ref):
    o_ref[...] = (
        jnp.dot(p_ref[...], w_ref[...], preferred_element_type=jnp.float32)
        + b_ref[...]
    )

  return pl.pallas_call(
      body,
      out_shape=jax.ShapeDtypeStruct((pooled.shape[0], NUM_CLS), jnp.float32),
  )(pooled, w_t, bias)


def kernel(text_ids, length, emb, W, b):
  del length
  ids_a = text_ids[:, :CW]
  ids_b = jnp.pad(text_ids[:, CW:], ((0, 0), (0, 2 * CW - SEQ)))
  pooled = _sc_pool(text_ids.shape[0])(ids_a, ids_b, emb)
  return _tc_head(pooled, W.T, b.reshape(1, NUM_CLS))
